# Initial kernel scaffold; baseline (speedup 1.0000x reference)
import jax
import jax.numpy as jnp
from jax.experimental import pallas as pl


def kernel(y, feat, centers):
    raise NotImplementedError("write your pallas kernel here")


# speedup vs baseline: 1.2682x; 1.2682x over previous
import functools

import jax
import jax.numpy as jnp
from jax import lax
from jax.experimental import pallas as pl
from jax.experimental.pallas import tpu as pltpu
from jax.experimental.pallas import tpu_sc as plsc

_B = 16384
_D = 128
_C = 100000
_CP = 102400
_HSLICE = _CP // 16
_NW = 32
_RPT = _B // _NW
_YROWS = _B // 128

_mesh = plsc.VectorSubcoreMesh(core_axis_name="c", subcore_axis_name="s")


@functools.partial(
    pl.kernel,
    mesh=_mesh,
    out_type=(
        jax.ShapeDtypeStruct((_B, _D), jnp.float32),
        jax.ShapeDtypeStruct((_YROWS, 128), jnp.float32),
    ),
    scratch_types=[
        pltpu.VMEM((8, 128), jnp.int32),
        pltpu.VMEM((4, 128), jnp.int32),
        pltpu.VMEM((128,), jnp.float32),
        pltpu.VMEM((_HSLICE,), jnp.float32),
        pltpu.VMEM((4, 128), jnp.float32),
        pltpu.VMEM((_RPT, _D), jnp.float32),
        pltpu.VMEM_SHARED((_CP,), jnp.float32),
    ],
)
def _sc_hist_gather(y2_hbm, centers_hbm, cp_hbm, cnt_hbm,
                    yh_v, yg_v, ones_v, zeros_v, cnt_v, rows_v, hist_sh):
    cid = lax.axis_index("c")
    sid = lax.axis_index("s")
    wid = sid * 2 + cid

    def _zfill(i, carry):
        zeros_v[pl.ds(i * 16, 16)] = jnp.zeros((16,), jnp.float32)
        return carry
    lax.fori_loop(0, _HSLICE // 16, _zfill, 0)
    for k in range(8):
        ones_v[pl.ds(k * 16, 16)] = jnp.ones((16,), jnp.float32)

    pltpu.sync_copy(zeros_v, hist_sh.at[pl.ds(sid * _HSLICE, _HSLICE)])

    pltpu.sync_copy(y2_hbm.at[pl.ds(sid * 8, 8)], yh_v)
    plsc.subcore_barrier()
    for j in range(8):
        pltpu.sync_copy(ones_v, hist_sh.at[yh_v.at[j]], add=True)
    plsc.subcore_barrier()

    pltpu.sync_copy(y2_hbm.at[pl.ds(wid * 4, 4)], yg_v)
    for j in range(4):
        pltpu.sync_copy(hist_sh.at[yg_v.at[j]], cnt_v.at[j])
        pltpu.sync_copy(centers_hbm.at[yg_v.at[j]],
                        rows_v.at[pl.ds(j * 128, 128)])
    pltpu.sync_copy(cnt_v, cnt_hbm.at[pl.ds(wid * 4, 4)])
    pltpu.sync_copy(rows_v, cp_hbm.at[pl.ds(wid * _RPT, _RPT)])


_ROWS_PER_STEP = 2048
_STEPS = _B // _ROWS_PER_STEP


def _tc_body(f_ref, c_ref, cnt_ref, out_ref, a0, a1, a2, a3, a4):
    i = pl.program_id(0)

    @pl.when(i == 0)
    def _init():
        for a in (a0, a1, a2, a3, a4):
            a[...] = jnp.zeros((1, _D), jnp.float32)

    f = f_ref[...]
    c = c_ref[...]
    w = 1.0 / (cnt_ref[...] + 1.0)
    ff = f * f
    cc = c * c
    fc = f * c
    a0[...] += jnp.sum(ff, axis=0, keepdims=True)
    a1[...] += jnp.sum(cc, axis=0, keepdims=True)
    a2[...] += jnp.sum(w * ff, axis=0, keepdims=True)
    a3[...] += jnp.sum(w * cc, axis=0, keepdims=True)
    a4[...] += jnp.sum(w * fc, axis=0, keepdims=True)

    @pl.when(i == _STEPS - 1)
    def _finish():
        nf = jnp.sqrt(a0[...])
        nc = jnp.sqrt(a1[...])
        sf = jnp.where(nf > 1e-5, 1e-5 / jnp.maximum(nf, 1e-30), 1.0) * 1e5
        sc = jnp.where(nc > 1e-5, 1e-5 / jnp.maximum(nc, 1e-30), 1.0) * 1e5
        val = sf * sf * a2[...] + sc * sc * a3[...] - 2.0 * (sf * sc) * a4[...]
        out_ref[...] = 0.5 * jnp.sum(val, axis=1, keepdims=True)


def _tc_loss(feat, cp, cnt):
    return pl.pallas_call(
        _tc_body,
        grid=(_STEPS,),
        in_specs=[
            pl.BlockSpec((_ROWS_PER_STEP, _D), lambda i: (i, 0)),
            pl.BlockSpec((_ROWS_PER_STEP, _D), lambda i: (i, 0)),
            pl.BlockSpec((_ROWS_PER_STEP, 1), lambda i: (i, 0)),
        ],
        out_specs=pl.BlockSpec((1, 1), lambda i: (0, 0)),
        out_shape=jax.ShapeDtypeStruct((1, 1), jnp.float32),
        scratch_shapes=[pltpu.VMEM((1, _D), jnp.float32)] * 5,
    )(feat, cp, cnt)


def kernel(y, feat, centers):
    y2 = y.astype(jnp.int32).reshape(_YROWS, 128)
    cp, cnt2 = _sc_hist_gather(y2, centers)
    loss = _tc_loss(feat, cp, cnt2.reshape(_B, 1))
    return loss[0, 0]


# speedup vs baseline: 1.4256x; 1.1241x over previous
import functools

import jax
import jax.numpy as jnp
from jax import lax
from jax.experimental import pallas as pl
from jax.experimental.pallas import tpu as pltpu
from jax.experimental.pallas import tpu_sc as plsc

_B = 16384
_D = 128
_C = 100000
_CP = 102400
_HSLICE = _CP // 16
_NW = 32
_RPT = _B // _NW
_YROWS = _B // 128

_mesh = plsc.VectorSubcoreMesh(core_axis_name="c", subcore_axis_name="s")


@functools.partial(
    pl.kernel,
    mesh=_mesh,
    out_type=(
        jax.ShapeDtypeStruct((_B, _D), jnp.float32),
        jax.ShapeDtypeStruct((_YROWS, 128), jnp.float32),
    ),
    scratch_types=[
        pltpu.VMEM((8, 128), jnp.int32),
        pltpu.VMEM((4, 128), jnp.int32),
        pltpu.VMEM((128,), jnp.float32),
        pltpu.VMEM((128,), jnp.float32),
        pltpu.VMEM((4, 128), jnp.float32),
        pltpu.VMEM((_RPT, _D), jnp.float32),
        pltpu.VMEM_SHARED((_CP,), jnp.float32),
        pltpu.SemaphoreType.DMA,
        pltpu.SemaphoreType.DMA,
        pltpu.SemaphoreType.DMA,
        pltpu.SemaphoreType.DMA,
    ],
)
def _sc_hist_gather(y2_hbm, centers_hbm, cp_hbm, cnt_hbm,
                    yh_v, yg_v, ones_v, zeros_v, cnt_v, rows_v, hist_sh,
                    semg, semh, semc, semw):
    cid = lax.axis_index("c")
    sid = lax.axis_index("s")
    wid = sid * 2 + cid

    pltpu.sync_copy(y2_hbm.at[pl.ds(wid * 4, 4)], yg_v)
    gathers = [
        pltpu.async_copy(centers_hbm.at[yg_v.at[j]],
                         rows_v.at[pl.ds(j * 128, 128)], semg)
        for j in range(4)
    ]

    pltpu.sync_copy(y2_hbm.at[pl.ds(sid * 8, 8)], yh_v)
    for k in range(8):
        ones_v[pl.ds(k * 16, 16)] = jnp.ones((16,), jnp.float32)
        zeros_v[pl.ds(k * 16, 16)] = jnp.zeros((16,), jnp.float32)
    zs = [pltpu.async_copy(zeros_v, hist_sh.at[yh_v.at[j]], semh)
          for j in range(8)]
    for z in zs:
        z.wait()
    plsc.subcore_barrier()
    adds = [pltpu.async_copy(ones_v, hist_sh.at[yh_v.at[j]], semh, add=True)
            for j in range(8)]
    for a in adds:
        a.wait()
    plsc.subcore_barrier()

    cnts = [pltpu.async_copy(hist_sh.at[yg_v.at[j]], cnt_v.at[j], semc)
            for j in range(4)]

    writes = []
    for j in range(4):
        gathers[j].wait()
        writes.append(pltpu.async_copy(
            rows_v.at[pl.ds(j * 128, 128)],
            cp_hbm.at[pl.ds(wid * _RPT + j * 128, 128)], semw))
    for c in cnts:
        c.wait()
    pltpu.sync_copy(cnt_v, cnt_hbm.at[pl.ds(wid * 4, 4)])
    for wdma in writes:
        wdma.wait()


_ROWS_PER_STEP = 2048
_STEPS = _B // _ROWS_PER_STEP


def _tc_body(f_ref, c_ref, cnt_ref, out_ref, a0, a1, a2, a3, a4):
    i = pl.program_id(0)

    @pl.when(i == 0)
    def _init():
        for a in (a0, a1, a2, a3, a4):
            a[...] = jnp.zeros((1, _D), jnp.float32)

    f = f_ref[...]
    c = c_ref[...]
    w = 1.0 / (cnt_ref[...] + 1.0)
    ff = f * f
    cc = c * c
    fc = f * c
    a0[...] += jnp.sum(ff, axis=0, keepdims=True)
    a1[...] += jnp.sum(cc, axis=0, keepdims=True)
    a2[...] += jnp.sum(w * ff, axis=0, keepdims=True)
    a3[...] += jnp.sum(w * cc, axis=0, keepdims=True)
    a4[...] += jnp.sum(w * fc, axis=0, keepdims=True)

    @pl.when(i == _STEPS - 1)
    def _finish():
        nf = jnp.sqrt(a0[...])
        nc = jnp.sqrt(a1[...])
        sf = jnp.where(nf > 1e-5, 1e-5 / jnp.maximum(nf, 1e-30), 1.0) * 1e5
        sc = jnp.where(nc > 1e-5, 1e-5 / jnp.maximum(nc, 1e-30), 1.0) * 1e5
        val = sf * sf * a2[...] + sc * sc * a3[...] - 2.0 * (sf * sc) * a4[...]
        out_ref[...] = 0.5 * jnp.sum(val, axis=1, keepdims=True)


def _tc_loss(feat, cp, cnt):
    return pl.pallas_call(
        _tc_body,
        grid=(_STEPS,),
        in_specs=[
            pl.BlockSpec((_ROWS_PER_STEP, _D), lambda i: (i, 0)),
            pl.BlockSpec((_ROWS_PER_STEP, _D), lambda i: (i, 0)),
            pl.BlockSpec((_ROWS_PER_STEP, 1), lambda i: (i, 0)),
        ],
        out_specs=pl.BlockSpec((1, 1), lambda i: (0, 0)),
        out_shape=jax.ShapeDtypeStruct((1, 1), jnp.float32),
        scratch_shapes=[pltpu.VMEM((1, _D), jnp.float32)] * 5,
    )(feat, cp, cnt)


def kernel(y, feat, centers):
    y2 = y.astype(jnp.int32).reshape(_YROWS, 128)
    cp, cnt2 = _sc_hist_gather(y2, centers)
    loss = _tc_loss(feat, cp, cnt2.reshape(_B, 1))
    return loss[0, 0]


# speedup vs baseline: 1.6990x; 1.1918x over previous
import functools

import jax
import jax.numpy as jnp
from jax import lax
from jax.experimental import pallas as pl
from jax.experimental.pallas import tpu as pltpu
from jax.experimental.pallas import tpu_sc as plsc

_B = 16384
_D = 128
_C = 100000
_CP = 102400
_HSLICE = _CP // 16
_NW = 32
_RPT = _B // _NW
_YROWS = _B // 128

_mesh = plsc.VectorSubcoreMesh(core_axis_name="c", subcore_axis_name="s")


@functools.partial(
    pl.kernel,
    mesh=_mesh,
    out_type=(
        jax.ShapeDtypeStruct((_B, _D), jnp.float32),
        jax.ShapeDtypeStruct((_YROWS, 128), jnp.float32),
    ),
    scratch_types=[
        pltpu.VMEM((8, 128), jnp.int32),
        pltpu.VMEM((4, 128), jnp.int32),
        pltpu.VMEM((128,), jnp.float32),
        pltpu.VMEM((128,), jnp.float32),
        pltpu.VMEM((4, 128), jnp.float32),
        pltpu.VMEM((_RPT, _D), jnp.float32),
        pltpu.VMEM_SHARED((_CP,), jnp.float32),
        pltpu.SemaphoreType.DMA,
        pltpu.SemaphoreType.DMA,
        pltpu.SemaphoreType.DMA,
        pltpu.SemaphoreType.DMA,
    ],
)
def _sc_hist_gather(y2_hbm, centers_hbm, cp_hbm, cnt_hbm,
                    yh_v, yg_v, ones_v, zeros_v, cnt_v, rows_v, hist_sh,
                    semg, semh, semc, semw):
    cid = lax.axis_index("c")
    sid = lax.axis_index("s")
    wid = sid * 2 + cid

    pltpu.sync_copy(y2_hbm.at[pl.ds(wid * 4, 4)], yg_v)
    gathers = [
        pltpu.async_copy(centers_hbm.at[yg_v.at[j]],
                         rows_v.at[pl.ds(j * 128, 128)], semg)
        for j in range(4)
    ]

    pltpu.sync_copy(y2_hbm.at[pl.ds(sid * 8, 8)], yh_v)
    for k in range(8):
        ones_v[pl.ds(k * 16, 16)] = jnp.ones((16,), jnp.float32)
        zeros_v[pl.ds(k * 16, 16)] = jnp.zeros((16,), jnp.float32)
    zs = [pltpu.async_copy(zeros_v, hist_sh.at[yh_v.at[j]], semh)
          for j in range(8)]
    for z in zs:
        z.wait()
    plsc.subcore_barrier()
    adds = [pltpu.async_copy(ones_v, hist_sh.at[yh_v.at[j]], semh, add=True)
            for j in range(8)]
    for a in adds:
        a.wait()
    plsc.subcore_barrier()

    cnts = [pltpu.async_copy(hist_sh.at[yg_v.at[j]], cnt_v.at[j], semc)
            for j in range(4)]

    writes = []
    for j in range(4):
        gathers[j].wait()
        writes.append(pltpu.async_copy(
            rows_v.at[pl.ds(j * 128, 128)],
            cp_hbm.at[pl.ds(wid * _RPT + j * 128, 128)], semw))
    for c in cnts:
        c.wait()
    pltpu.sync_copy(cnt_v, cnt_hbm.at[pl.ds(wid * 4, 4)])
    for wdma in writes:
        wdma.wait()


_ROWS_PER_STEP = 2048
_STEPS = _B // _ROWS_PER_STEP


def _tc_body(f_ref, c_ref, cnt_ref, out_ref, a0, a1, a2, a3, a4):
    i = pl.program_id(0)

    @pl.when(i == 0)
    def _init():
        for a in (a0, a1, a2, a3, a4):
            a[...] = jnp.zeros((1, _D), jnp.float32)

    f = f_ref[...]
    c = c_ref[...]
    w16 = 1.0 / (cnt_ref[...] + 1.0)
    ff = f * f
    cc = c * c
    fc = f * c
    a0[...] += jnp.sum(ff, axis=0, keepdims=True)
    a1[...] += jnp.sum(cc, axis=0, keepdims=True)
    p = jnp.zeros((1, _D), jnp.float32)
    q = jnp.zeros((1, _D), jnp.float32)
    r = jnp.zeros((1, _D), jnp.float32)
    for k in range(_ROWS_PER_STEP // 128):
        wk = w16[k:k + 1, :]
        sl = slice(k * 128, (k + 1) * 128)
        p += jax.lax.dot(wk, ff[sl, :],
                         preferred_element_type=jnp.float32)
        q += jax.lax.dot(wk, cc[sl, :],
                         preferred_element_type=jnp.float32)
        r += jax.lax.dot(wk, fc[sl, :],
                         preferred_element_type=jnp.float32)
    a2[...] += p
    a3[...] += q
    a4[...] += r

    @pl.when(i == _STEPS - 1)
    def _finish():
        nf = jnp.sqrt(a0[...])
        nc = jnp.sqrt(a1[...])
        sf = jnp.where(nf > 1e-5, 1e-5 / jnp.maximum(nf, 1e-30), 1.0) * 1e5
        sc = jnp.where(nc > 1e-5, 1e-5 / jnp.maximum(nc, 1e-30), 1.0) * 1e5
        val = sf * sf * a2[...] + sc * sc * a3[...] - 2.0 * (sf * sc) * a4[...]
        out_ref[...] = 0.5 * jnp.sum(val, axis=1, keepdims=True)


def _tc_loss(feat, cp, cnt):
    return pl.pallas_call(
        _tc_body,
        grid=(_STEPS,),
        in_specs=[
            pl.BlockSpec((_ROWS_PER_STEP, _D), lambda i: (i, 0)),
            pl.BlockSpec((_ROWS_PER_STEP, _D), lambda i: (i, 0)),
            pl.BlockSpec((_ROWS_PER_STEP // 128, 128), lambda i: (i, 0)),
        ],
        out_specs=pl.BlockSpec((1, 1), lambda i: (0, 0)),
        out_shape=jax.ShapeDtypeStruct((1, 1), jnp.float32),
        scratch_shapes=[pltpu.VMEM((1, _D), jnp.float32)] * 5,
    )(feat, cp, cnt)


def kernel(y, feat, centers):
    y2 = y.astype(jnp.int32).reshape(_YROWS, 128)
    cp, cnt2 = _sc_hist_gather(y2, centers)
    loss = _tc_loss(feat, cp, cnt2)
    return loss[0, 0]


# speedup vs baseline: 1.7880x; 1.0524x over previous
import functools

import jax
import jax.numpy as jnp
from jax import lax
from jax.experimental import pallas as pl
from jax.experimental.pallas import tpu as pltpu
from jax.experimental.pallas import tpu_sc as plsc

_B = 16384
_D = 128
_C = 100000
_CP = 102400
_HSLICE = _CP // 16
_NW = 32
_RPT = _B // _NW
_YROWS = _B // 128

_mesh = plsc.VectorSubcoreMesh(core_axis_name="c", subcore_axis_name="s")


@functools.partial(
    pl.kernel,
    mesh=_mesh,
    out_type=(
        jax.ShapeDtypeStruct((_B, _D), jnp.float32),
        jax.ShapeDtypeStruct((_YROWS, 128), jnp.float32),
    ),
    scratch_types=[
        pltpu.VMEM((8, 128), jnp.int32),
        pltpu.VMEM((4, 128), jnp.int32),
        pltpu.VMEM((128,), jnp.float32),
        pltpu.VMEM((128,), jnp.float32),
        pltpu.VMEM((4, 128), jnp.float32),
        pltpu.VMEM((_RPT, _D), jnp.float32),
        pltpu.VMEM_SHARED((_CP,), jnp.float32),
        pltpu.SemaphoreType.DMA,
        pltpu.SemaphoreType.DMA,
        pltpu.SemaphoreType.DMA,
        pltpu.SemaphoreType.DMA,
    ],
)
def _sc_hist_gather(y2_hbm, centers_hbm, cp_hbm, cnt_hbm,
                    yh_v, yg_v, ones_v, zeros_v, cnt_v, rows_v, hist_sh,
                    semg, semh, semc, semw):
    cid = lax.axis_index("c")
    sid = lax.axis_index("s")
    wid = sid * 2 + cid

    pltpu.sync_copy(y2_hbm.at[pl.ds(wid * 4, 4)], yg_v)
    gathers = [
        pltpu.async_copy(centers_hbm.at[yg_v.at[j]],
                         rows_v.at[pl.ds(j * 128, 128)], semg)
        for j in range(4)
    ]

    pltpu.sync_copy(y2_hbm.at[pl.ds(sid * 8, 8)], yh_v)
    for k in range(8):
        ones_v[pl.ds(k * 16, 16)] = jnp.ones((16,), jnp.float32)
        zeros_v[pl.ds(k * 16, 16)] = jnp.zeros((16,), jnp.float32)
    zs = [pltpu.async_copy(zeros_v, hist_sh.at[yh_v.at[j]], semh)
          for j in range(8)]
    for z in zs:
        z.wait()
    plsc.subcore_barrier()
    adds = [pltpu.async_copy(ones_v, hist_sh.at[yh_v.at[j]], semh, add=True)
            for j in range(8)]
    for a in adds:
        a.wait()
    plsc.subcore_barrier()

    cnts = [pltpu.async_copy(hist_sh.at[yg_v.at[j]], cnt_v.at[j], semc)
            for j in range(4)]

    writes = []
    for j in range(4):
        gathers[j].wait()
        writes.append(pltpu.async_copy(
            rows_v.at[pl.ds(j * 128, 128)],
            cp_hbm.at[pl.ds(wid * _RPT + j * 128, 128)], semw))
    for c in cnts:
        c.wait()
    pltpu.sync_copy(cnt_v, cnt_hbm.at[pl.ds(wid * 4, 4)])
    for wdma in writes:
        wdma.wait()


_ROWS_PER_STEP = 4096
_STEPS = _B // _ROWS_PER_STEP


def _tc_body(f_ref, c_ref, cnt_ref, out_ref, a0, a1, a2, a3, a4):
    i = pl.program_id(0)

    @pl.when(i == 0)
    def _init():
        for a in (a0, a1, a2, a3, a4):
            a[...] = jnp.zeros((1, _D), jnp.float32)

    f = f_ref[...]
    c = c_ref[...]
    w16 = 1.0 / (cnt_ref[...] + 1.0)
    ff = f * f
    cc = c * c
    fc = f * c
    a0[...] += jnp.sum(ff, axis=0, keepdims=True)
    a1[...] += jnp.sum(cc, axis=0, keepdims=True)
    p = jnp.zeros((1, _D), jnp.float32)
    q = jnp.zeros((1, _D), jnp.float32)
    r = jnp.zeros((1, _D), jnp.float32)
    for k in range(_ROWS_PER_STEP // 128):
        wk = w16[k:k + 1, :]
        sl = slice(k * 128, (k + 1) * 128)
        p += jax.lax.dot(wk, ff[sl, :],
                         preferred_element_type=jnp.float32)
        q += jax.lax.dot(wk, cc[sl, :],
                         preferred_element_type=jnp.float32)
        r += jax.lax.dot(wk, fc[sl, :],
                         preferred_element_type=jnp.float32)
    a2[...] += p
    a3[...] += q
    a4[...] += r

    @pl.when(i == _STEPS - 1)
    def _finish():
        nf = jnp.sqrt(a0[...])
        nc = jnp.sqrt(a1[...])
        sf = jnp.where(nf > 1e-5, 1e-5 / jnp.maximum(nf, 1e-30), 1.0) * 1e5
        sc = jnp.where(nc > 1e-5, 1e-5 / jnp.maximum(nc, 1e-30), 1.0) * 1e5
        val = sf * sf * a2[...] + sc * sc * a3[...] - 2.0 * (sf * sc) * a4[...]
        out_ref[...] = 0.5 * jnp.sum(val, axis=1, keepdims=True)


def _tc_loss(feat, cp, cnt):
    return pl.pallas_call(
        _tc_body,
        grid=(_STEPS,),
        in_specs=[
            pl.BlockSpec((_ROWS_PER_STEP, _D), lambda i: (i, 0)),
            pl.BlockSpec((_ROWS_PER_STEP, _D), lambda i: (i, 0)),
            pl.BlockSpec((_ROWS_PER_STEP // 128, 128), lambda i: (i, 0)),
        ],
        out_specs=pl.BlockSpec((1, 1), lambda i: (0, 0)),
        out_shape=jax.ShapeDtypeStruct((1, 1), jnp.float32),
        scratch_shapes=[pltpu.VMEM((1, _D), jnp.float32)] * 5,
    )(feat, cp, cnt)


def kernel(y, feat, centers):
    y2 = y.astype(jnp.int32).reshape(_YROWS, 128)
    cp, cnt2 = _sc_hist_gather(y2, centers)
    loss = _tc_loss(feat, cp, cnt2)
    return loss[0, 0]


# speedup vs baseline: 1.7898x; 1.0010x over previous
import functools

import jax
import jax.numpy as jnp
from jax import lax
from jax.experimental import pallas as pl
from jax.experimental.pallas import tpu as pltpu
from jax.experimental.pallas import tpu_sc as plsc

_B = 16384
_D = 128
_C = 100000
_CP = 102400
_HSLICE = _CP // 16
_NW = 32
_RPT = _B // _NW
_YROWS = _B // 128

_mesh = plsc.VectorSubcoreMesh(core_axis_name="c", subcore_axis_name="s")


@functools.partial(
    pl.kernel,
    mesh=_mesh,
    out_type=(
        jax.ShapeDtypeStruct((_B, _D), jnp.float32),
        jax.ShapeDtypeStruct((_YROWS, 128), jnp.float32),
    ),
    scratch_types=[
        pltpu.VMEM((8, 128), jnp.int32),
        pltpu.VMEM((4, 128), jnp.int32),
        pltpu.VMEM((128,), jnp.float32),
        pltpu.VMEM((128,), jnp.float32),
        pltpu.VMEM((4, 128), jnp.float32),
        pltpu.VMEM((_RPT, _D), jnp.float32),
        pltpu.VMEM_SHARED((_CP,), jnp.float32),
        pltpu.SemaphoreType.DMA,
        pltpu.SemaphoreType.DMA,
        pltpu.SemaphoreType.DMA,
        pltpu.SemaphoreType.DMA,
    ],
)
def _sc_hist_gather(y2_hbm, centers_hbm, cp_hbm, cnt_hbm,
                    yh_v, yg_v, ones_v, zeros_v, cnt_v, rows_v, hist_sh,
                    semg, semh, semc, semw):
    cid = lax.axis_index("c")
    sid = lax.axis_index("s")
    wid = sid * 2 + cid

    pltpu.sync_copy(y2_hbm.at[pl.ds(wid * 4, 4)], yg_v)
    gathers = [
        pltpu.async_copy(centers_hbm.at[yg_v.at[j]],
                         rows_v.at[pl.ds(j * 128, 128)], semg)
        for j in range(4)
    ]

    pltpu.sync_copy(y2_hbm.at[pl.ds(sid * 8, 8)], yh_v)
    for k in range(8):
        ones_v[pl.ds(k * 16, 16)] = jnp.ones((16,), jnp.float32)
        zeros_v[pl.ds(k * 16, 16)] = jnp.zeros((16,), jnp.float32)
    zs = [pltpu.async_copy(zeros_v, hist_sh.at[yh_v.at[j]], semh)
          for j in range(8)]
    for z in zs:
        z.wait()
    plsc.subcore_barrier()
    adds = [pltpu.async_copy(ones_v, hist_sh.at[yh_v.at[j]], semh, add=True)
            for j in range(8)]
    for a in adds:
        a.wait()
    plsc.subcore_barrier()

    cnts = [pltpu.async_copy(hist_sh.at[yg_v.at[j]], cnt_v.at[j], semc)
            for j in range(4)]

    writes = []
    for j in range(4):
        gathers[j].wait()
        writes.append(pltpu.async_copy(
            rows_v.at[pl.ds(j * 128, 128)],
            cp_hbm.at[pl.ds(wid * _RPT + j * 128, 128)], semw))
    for c in cnts:
        c.wait()
    pltpu.sync_copy(cnt_v, cnt_hbm.at[pl.ds(wid * 4, 4)])
    for wdma in writes:
        wdma.wait()


_ROWS_PER_STEP = 8192
_STEPS = _B // _ROWS_PER_STEP


def _tc_body(f_ref, c_ref, cnt_ref, out_ref, a0, a1, a2, a3, a4):
    i = pl.program_id(0)

    @pl.when(i == 0)
    def _init():
        for a in (a0, a1, a2, a3, a4):
            a[...] = jnp.zeros((1, _D), jnp.float32)

    f = f_ref[...]
    c = c_ref[...]
    w16 = 1.0 / (cnt_ref[...] + 1.0)
    ff = f * f
    cc = c * c
    fc = f * c
    a0[...] += jnp.sum(ff, axis=0, keepdims=True)
    a1[...] += jnp.sum(cc, axis=0, keepdims=True)
    p = jnp.zeros((1, _D), jnp.float32)
    q = jnp.zeros((1, _D), jnp.float32)
    r = jnp.zeros((1, _D), jnp.float32)
    for k in range(_ROWS_PER_STEP // 128):
        wk = w16[k:k + 1, :]
        sl = slice(k * 128, (k + 1) * 128)
        p += jax.lax.dot(wk, ff[sl, :],
                         preferred_element_type=jnp.float32)
        q += jax.lax.dot(wk, cc[sl, :],
                         preferred_element_type=jnp.float32)
        r += jax.lax.dot(wk, fc[sl, :],
                         preferred_element_type=jnp.float32)
    a2[...] += p
    a3[...] += q
    a4[...] += r

    @pl.when(i == _STEPS - 1)
    def _finish():
        nf = jnp.sqrt(a0[...])
        nc = jnp.sqrt(a1[...])
        sf = jnp.where(nf > 1e-5, 1e-5 / jnp.maximum(nf, 1e-30), 1.0) * 1e5
        sc = jnp.where(nc > 1e-5, 1e-5 / jnp.maximum(nc, 1e-30), 1.0) * 1e5
        val = sf * sf * a2[...] + sc * sc * a3[...] - 2.0 * (sf * sc) * a4[...]
        out_ref[...] = 0.5 * jnp.sum(val, axis=1, keepdims=True)


def _tc_loss(feat, cp, cnt):
    return pl.pallas_call(
        _tc_body,
        grid=(_STEPS,),
        in_specs=[
            pl.BlockSpec((_ROWS_PER_STEP, _D), lambda i: (i, 0)),
            pl.BlockSpec((_ROWS_PER_STEP, _D), lambda i: (i, 0)),
            pl.BlockSpec((_ROWS_PER_STEP // 128, 128), lambda i: (i, 0)),
        ],
        out_specs=pl.BlockSpec((1, 1), lambda i: (0, 0)),
        out_shape=jax.ShapeDtypeStruct((1, 1), jnp.float32),
        scratch_shapes=[pltpu.VMEM((1, _D), jnp.float32)] * 5,
    )(feat, cp, cnt)


def kernel(y, feat, centers):
    y2 = y.astype(jnp.int32).reshape(_YROWS, 128)
    cp, cnt2 = _sc_hist_gather(y2, centers)
    loss = _tc_loss(feat, cp, cnt2)
    return loss[0, 0]


# speedup vs baseline: 1.7898x; 1.0000x over previous
import functools

import jax
import jax.numpy as jnp
from jax import lax
from jax.experimental import pallas as pl
from jax.experimental.pallas import tpu as pltpu
from jax.experimental.pallas import tpu_sc as plsc

_B = 16384
_D = 128
_C = 100000
_CP = 102400
_HSLICE = _CP // 16
_NW = 32
_RPT = _B // _NW
_YROWS = _B // 128

_mesh = plsc.VectorSubcoreMesh(core_axis_name="c", subcore_axis_name="s")


@functools.partial(
    pl.kernel,
    mesh=_mesh,
    out_type=(
        jax.ShapeDtypeStruct((_B, _D), jnp.float32),
        jax.ShapeDtypeStruct((_YROWS, 128), jnp.float32),
    ),
    scratch_types=[
        pltpu.VMEM((8, 128), jnp.int32),
        pltpu.VMEM((4, 128), jnp.int32),
        pltpu.VMEM((128,), jnp.float32),
        pltpu.VMEM((128,), jnp.float32),
        pltpu.VMEM((4, 128), jnp.float32),
        pltpu.VMEM((_RPT, _D), jnp.float32),
        pltpu.VMEM_SHARED((_CP,), jnp.float32),
        pltpu.SemaphoreType.DMA,
        pltpu.SemaphoreType.DMA,
        pltpu.SemaphoreType.DMA,
        pltpu.SemaphoreType.DMA,
    ],
)
def _sc_hist_gather(y2_hbm, centers_hbm, cp_hbm, cnt_hbm,
                    yh_v, yg_v, ones_v, zeros_v, cnt_v, rows_v, hist_sh,
                    semg, semh, semc, semw):
    cid = lax.axis_index("c")
    sid = lax.axis_index("s")
    wid = sid * 2 + cid

    pltpu.sync_copy(y2_hbm.at[pl.ds(wid * 4, 4)], yg_v)
    gathers = [
        pltpu.async_copy(centers_hbm.at[yg_v.at[j]],
                         rows_v.at[pl.ds(j * 128, 128)], semg)
        for j in range(4)
    ]

    pltpu.sync_copy(y2_hbm.at[pl.ds(sid * 8, 8)], yh_v)
    for k in range(8):
        ones_v[pl.ds(k * 16, 16)] = jnp.ones((16,), jnp.float32)
        zeros_v[pl.ds(k * 16, 16)] = jnp.zeros((16,), jnp.float32)
    zs = [pltpu.async_copy(zeros_v, hist_sh.at[yh_v.at[j]], semh)
          for j in range(8)]
    for z in zs:
        z.wait()
    plsc.subcore_barrier()
    adds = [pltpu.async_copy(ones_v, hist_sh.at[yh_v.at[j]], semh, add=True)
            for j in range(8)]
    for a in adds:
        a.wait()
    plsc.subcore_barrier()

    cnts = [pltpu.async_copy(hist_sh.at[yg_v.at[j]], cnt_v.at[j], semc)
            for j in range(4)]

    writes = []
    for j in range(4):
        gathers[j].wait()
        writes.append(pltpu.async_copy(
            rows_v.at[pl.ds(j * 128, 128)],
            cp_hbm.at[pl.ds(wid * _RPT + j * 128, 128)], semw))
    for c in cnts:
        c.wait()
    pltpu.sync_copy(cnt_v, cnt_hbm.at[pl.ds(wid * 4, 4)])
    for wdma in writes:
        wdma.wait()


_ROWS_PER_STEP = 4096
_STEPS = _B // _ROWS_PER_STEP


def _tc_body(f_ref, c_ref, cnt_ref, out_ref, a0, a1, a2, a3, a4):
    i = pl.program_id(0)

    @pl.when(i == 0)
    def _init():
        for a in (a0, a1, a2, a3, a4):
            a[...] = jnp.zeros((1, _D), jnp.float32)

    f = f_ref[...]
    c = c_ref[...]
    w16 = 1.0 / (cnt_ref[...] + 1.0)
    ff = f * f
    cc = c * c
    fc = f * c
    a0[...] += jnp.sum(ff, axis=0, keepdims=True)
    a1[...] += jnp.sum(cc, axis=0, keepdims=True)
    p = jnp.zeros((1, _D), jnp.float32)
    q = jnp.zeros((1, _D), jnp.float32)
    r = jnp.zeros((1, _D), jnp.float32)
    for k in range(_ROWS_PER_STEP // 128):
        wk = w16[k:k + 1, :]
        sl = slice(k * 128, (k + 1) * 128)
        p += jax.lax.dot(wk, ff[sl, :],
                         preferred_element_type=jnp.float32)
        q += jax.lax.dot(wk, cc[sl, :],
                         preferred_element_type=jnp.float32)
        r += jax.lax.dot(wk, fc[sl, :],
                         preferred_element_type=jnp.float32)
    a2[...] += p
    a3[...] += q
    a4[...] += r

    @pl.when(i == _STEPS - 1)
    def _finish():
        nf = jnp.sqrt(a0[...])
        nc = jnp.sqrt(a1[...])
        sf = jnp.where(nf > 1e-5, 1e-5 / jnp.maximum(nf, 1e-30), 1.0) * 1e5
        sc = jnp.where(nc > 1e-5, 1e-5 / jnp.maximum(nc, 1e-30), 1.0) * 1e5
        val = sf * sf * a2[...] + sc * sc * a3[...] - 2.0 * (sf * sc) * a4[...]
        out_ref[...] = 0.5 * jnp.sum(val, axis=1, keepdims=True)


def _tc_loss(feat, cp, cnt):
    return pl.pallas_call(
        _tc_body,
        grid=(_STEPS,),
        in_specs=[
            pl.BlockSpec((_ROWS_PER_STEP, _D), lambda i: (i, 0)),
            pl.BlockSpec((_ROWS_PER_STEP, _D), lambda i: (i, 0)),
            pl.BlockSpec((_ROWS_PER_STEP // 128, 128), lambda i: (i, 0)),
        ],
        out_specs=pl.BlockSpec((1, 1), lambda i: (0, 0)),
        out_shape=jax.ShapeDtypeStruct((1, 1), jnp.float32),
        scratch_shapes=[pltpu.VMEM((1, _D), jnp.float32)] * 5,
    )(feat, cp, cnt)


def kernel(y, feat, centers):
    y2 = y.astype(jnp.int32).reshape(_YROWS, 128)
    cp, cnt2 = _sc_hist_gather(y2, centers)
    loss = _tc_loss(feat, cp, cnt2)
    return loss[0, 0]
